# baseline (speedup 1.0000x reference)
import functools

import jax
import jax.numpy as jnp
from jax import lax
from jax.experimental import pallas as pl
from jax.experimental.pallas import tpu as pltpu
from jax.experimental.pallas import tpu_sc as plsc

N = 10000
E = 160000
D = 256

NC = 2
NS = 16
LANES = 16

EB = 128
NB = 80
EPT = NB * EB
EPAD = NS * EPT
DUMMY = N
NACC = N + 8
WT = 10

_MESH = plsc.VectorSubcoreMesh(
    core_axis_name="c", subcore_axis_name="s", num_cores=NC, num_subcores=NS
)


def _deg_body(colp, ones_hbm, zer_hbm, deg_out, dacc, ones_v, cidx, dsem):
    c = lax.axis_index("c")
    s = lax.axis_index("s")

    pltpu.sync_copy(ones_hbm, ones_v)
    pltpu.sync_copy(colp.at[s], cidx)

    @pl.when(s < WT)
    def _():
        pltpu.sync_copy(zer_hbm, dacc.at[pl.ds(s * 1000, 1000)])

    plsc.subcore_barrier()

    KF = 8
    b0 = c * (NB // 2)

    def round_(r, carry):
        def fire(i, cc):
            pltpu.async_copy(ones_v, dacc.at[cidx.at[b0 + r * KF + i]], dsem,
                             add=True)
            return cc

        lax.fori_loop(0, KF, fire, 0)

        def drain(i, cc):
            pltpu.make_async_copy(ones_hbm, ones_v, dsem).wait()
            return cc

        lax.fori_loop(0, KF, drain, 0)
        return carry

    lax.fori_loop(0, (NB // 2) // KF, round_, 0)
    plsc.subcore_barrier()

    @pl.when(s < WT)
    def _():
        pltpu.sync_copy(dacc.at[pl.ds(s * 1000, 1000)],
                        deg_out.at[c, pl.ds(s * 1000, 1000)])


@functools.partial(
    pl.kernel,
    out_type=jax.ShapeDtypeStruct((NC, N, 128), jnp.float32),
    mesh=_MESH,
    scratch_types=[
        pltpu.VMEM_SHARED((NACC, 128), jnp.float32),
        pltpu.VMEM((EB, 128), jnp.float32),
        pltpu.VMEM((NB, EB), jnp.int32),
        pltpu.SemaphoreType.DMA,
    ],
)
def _deg_kernel(colp, ones_hbm, zer_hbm, deg_out, dacc, ones_v, cidx, dsem):
    _deg_body(colp, ones_hbm, zer_hbm, deg_out, dacc, ones_v, cidx, dsem)


CHK = 16
NCHK = NB // CHK


def _agg_body(g2, rowp, colp, zer_hbm, out, acc, ridx, cidx2, gbuf,
              gsems, ssems, csems):
    c = lax.axis_index("c")
    s = lax.axis_index("s")

    pltpu.sync_copy(rowp.at[s], ridx)

    @pl.when(s < WT)
    def _():
        pltpu.sync_copy(zer_hbm, acc.at[pl.ds(s * 1000, 1000)])

    def xform(r, carry):
        for j in range(EB // LANES):
            v = ridx[r, pl.ds(j * LANES, LANES)]
            ridx[r, pl.ds(j * LANES, LANES)] = v * 2 + c
        return carry

    lax.fori_loop(0, NB + 1, xform, 0)
    plsc.subcore_barrier()

    def wait_dma(dst, sem):
        pltpu.make_async_copy(g2.at[pl.ds(0, EB)], dst, sem).wait()

    pltpu.async_copy(g2.at[ridx.at[0]], gbuf.at[0], gsems[0])
    pltpu.async_copy(colp.at[s, pl.ds(0, CHK)], cidx2.at[0], csems[0])

    def step(b, carry):
        q = b // CHK

        @pl.when(lax.rem(b, CHK) == 0)
        def _():
            for t in range(2):
                @pl.when(lax.rem(q, 2) == t)
                def _():
                    pltpu.make_async_copy(colp.at[s, pl.ds(0, CHK)],
                                          cidx2.at[t], csems[t]).wait()

                    @pl.when(q + 1 < NCHK)
                    def _():
                        pltpu.async_copy(
                            colp.at[s, pl.ds((q + 1) * CHK, CHK)],
                            cidx2.at[1 - t], csems[1 - t])

        for p in range(2):
            @pl.when(lax.rem(b, 2) == p)
            def _():
                wait_dma(gbuf.at[p], gsems[p])
                pltpu.async_copy(
                    gbuf.at[p], acc.at[cidx2.at[lax.rem(q, 2), lax.rem(b, CHK)]],
                    ssems[p], add=True)

                @pl.when(b >= 1)
                def _():
                    wait_dma(gbuf.at[1 - p], ssems[1 - p])

                pltpu.async_copy(g2.at[ridx.at[b + 1]], gbuf.at[1 - p],
                                 gsems[1 - p])
        return carry

    lax.fori_loop(0, NB, step, 0)

    wait_dma(gbuf.at[(NB - 1) % 2], ssems[(NB - 1) % 2])
    wait_dma(gbuf.at[NB % 2], gsems[NB % 2])

    plsc.subcore_barrier()

    @pl.when(s < WT)
    def _():
        pltpu.sync_copy(acc.at[pl.ds(s * 1000, 1000)],
                        out.at[c, pl.ds(s * 1000, 1000)])


@functools.partial(
    pl.kernel,
    out_type=jax.ShapeDtypeStruct((NC, N, 128), jnp.float32),
    mesh=_MESH,
    scratch_types=[
        pltpu.VMEM_SHARED((NACC, 128), jnp.float32),
        pltpu.VMEM((NB + 1, EB), jnp.int32),
        pltpu.VMEM((2, CHK, EB), jnp.int32),
        pltpu.VMEM((2, EB, 128), jnp.float32),
        [pltpu.SemaphoreType.DMA] * 2,
        [pltpu.SemaphoreType.DMA] * 2,
        [pltpu.SemaphoreType.DMA] * 2,
    ],
)
def _agg_kernel(g2, rowp, colp, zer_hbm, out, acc, ridx, cidx2, gbuf,
                gsems, ssems, csems):
    _agg_body(g2, rowp, colp, zer_hbm, out, acc, ridx, cidx2, gbuf,
              gsems, ssems, csems)


_BT = 1000


def _prep_body(deg_ref, x_ref, g_ref):
    dinv = lax.rsqrt(deg_ref[0, :, 0:1] + deg_ref[1, :, 0:1] + 1.0)
    g_ref[...] = dinv * x_ref[...]


def _layer_body(relu, emit_g, deg_ref, s_ref, g_ref, res_ref, w_ref, *outs):
    dinv = lax.rsqrt(deg_ref[0, :, 0:1] + deg_ref[1, :, 0:1] + 1.0)
    sfull = jnp.concatenate([s_ref[0], s_ref[1]], axis=-1)
    agg = dinv * (sfull + g_ref[...])
    hv = lax.dot_general(agg, w_ref[...], (((1,), (1,)), ((), ())),
                         preferred_element_type=jnp.float32)
    if relu:
        hv = jnp.maximum(hv, 0.0)
    hv = hv + res_ref[...]
    outs[0][...] = hv
    if emit_g:
        outs[1][...] = dinv * hv


def _tc_prep(deg2, x):
    return pl.pallas_call(
        _prep_body,
        grid=(N // _BT,),
        in_specs=[
            pl.BlockSpec((NC, _BT, 128), lambda i: (0, i, 0)),
            pl.BlockSpec((_BT, D), lambda i: (i, 0)),
        ],
        out_specs=pl.BlockSpec((_BT, D), lambda i: (i, 0)),
        out_shape=jax.ShapeDtypeStruct((N, D), jnp.float32),
    )(deg2, x)


def _tc_layer(deg2, s2, g, res, w, relu, emit_g):
    n_out = 2 if emit_g else 1
    out_shape = [jax.ShapeDtypeStruct((N, D), jnp.float32)] * n_out
    out_specs = [pl.BlockSpec((_BT, D), lambda i: (i, 0))] * n_out
    res_ = pl.pallas_call(
        functools.partial(_layer_body, relu, emit_g),
        grid=(N // _BT,),
        in_specs=[
            pl.BlockSpec((NC, _BT, 128), lambda i: (0, i, 0)),
            pl.BlockSpec((NC, _BT, 128), lambda i: (0, i, 0)),
            pl.BlockSpec((_BT, D), lambda i: (i, 0)),
            pl.BlockSpec((_BT, D), lambda i: (i, 0)),
            pl.BlockSpec((D, D), lambda i: (0, 0)),
        ],
        out_specs=out_specs,
        out_shape=out_shape,
    )(deg2, s2, g, res, w)
    return res_ if emit_g else res_[0]


def kernel(x, edge_index, W0, W1):
    row = edge_index[0]
    col = edge_index[1]

    rowp = jnp.concatenate(
        [row, jnp.zeros((EPAD - E,), jnp.int32)]).reshape(NS, NB, EB)
    rowp = jnp.concatenate([rowp, jnp.zeros((NS, 1, EB), jnp.int32)], axis=1)
    colp = jnp.concatenate(
        [col, jnp.full((EPAD - E,), DUMMY, jnp.int32)]).reshape(NS, NB, EB)

    ones_hbm = jnp.ones((EB, 128), jnp.float32)
    zer128 = jnp.zeros((1000, 128), jnp.float32)

    deg2 = _deg_kernel(colp, ones_hbm, zer128)

    g0 = _tc_prep(deg2, x)
    s0 = _agg_kernel(g0.reshape(2 * N, 128), rowp, colp, zer128)
    h1, g1 = _tc_layer(deg2, s0, g0, x, W0, relu=True, emit_g=True)
    s1 = _agg_kernel(g1.reshape(2 * N, 128), rowp, colp, zer128)
    h2 = _tc_layer(deg2, s1, g1, h1, W1, relu=False, emit_g=False)
    return h2[None]


# speedup vs baseline: 1.0789x; 1.0789x over previous
import functools

import jax
import jax.numpy as jnp
from jax import lax
from jax.experimental import pallas as pl
from jax.experimental.pallas import tpu as pltpu
from jax.experimental.pallas import tpu_sc as plsc

N = 10000
E = 160000
D = 256

NC = 2
NS = 16
LANES = 16

EB = 128
NB = 80
EPT = NB * EB
EPAD = NS * EPT
DUMMY = N
NACC = N + 8
WT = 10

_MESH = plsc.VectorSubcoreMesh(
    core_axis_name="c", subcore_axis_name="s", num_cores=NC, num_subcores=NS
)


def _deg_body(colp, ones_hbm, zer_hbm, deg_out, dacc, ones_v, cidx, dsem):
    c = lax.axis_index("c")
    s = lax.axis_index("s")

    pltpu.sync_copy(ones_hbm, ones_v)
    pltpu.sync_copy(colp.at[s], cidx)

    @pl.when(s < WT)
    def _():
        pltpu.sync_copy(zer_hbm, dacc.at[pl.ds(s * 1000, 1000)])

    plsc.subcore_barrier()

    KF = 8
    b0 = c * (NB // 2)

    def round_(r, carry):
        def fire(i, cc):
            pltpu.async_copy(ones_v, dacc.at[cidx.at[b0 + r * KF + i]], dsem,
                             add=True)
            return cc

        lax.fori_loop(0, KF, fire, 0)

        def drain(i, cc):
            pltpu.make_async_copy(ones_hbm, ones_v, dsem).wait()
            return cc

        lax.fori_loop(0, KF, drain, 0)
        return carry

    lax.fori_loop(0, (NB // 2) // KF, round_, 0)
    plsc.subcore_barrier()

    @pl.when(s < WT)
    def _():
        pltpu.sync_copy(dacc.at[pl.ds(s * 1000, 1000)],
                        deg_out.at[c, pl.ds(s * 1000, 1000)])


@functools.partial(
    pl.kernel,
    out_type=jax.ShapeDtypeStruct((NC, N, 128), jnp.float32),
    mesh=_MESH,
    scratch_types=[
        pltpu.VMEM_SHARED((NACC, 128), jnp.float32),
        pltpu.VMEM((EB, 128), jnp.float32),
        pltpu.VMEM((NB, EB), jnp.int32),
        pltpu.SemaphoreType.DMA,
    ],
)
def _deg_kernel(colp, ones_hbm, zer_hbm, deg_out, dacc, ones_v, cidx, dsem):
    _deg_body(colp, ones_hbm, zer_hbm, deg_out, dacc, ones_v, cidx, dsem)


def _agg_body(g2, rowp, colp, zer_hbm, out, acc, ridx, cidx, gbuf):
    c = lax.axis_index("c")
    s = lax.axis_index("s")

    pltpu.sync_copy(rowp.at[s], ridx)
    pltpu.sync_copy(colp.at[s], cidx)

    @pl.when(s < WT)
    def _():
        pltpu.sync_copy(zer_hbm, acc.at[pl.ds(s * 1000, 1000)])

    def xform(r, carry):
        for j in range(EB // LANES):
            v = ridx[r, pl.ds(j * LANES, LANES)]
            ridx[r, pl.ds(j * LANES, LANES)] = v * 2 + c
        return carry

    lax.fori_loop(0, NB + 1, xform, 0)
    plsc.subcore_barrier()

    def step(b, carry):
        pltpu.sync_copy(g2.at[ridx.at[b]], gbuf)
        pltpu.sync_copy(gbuf, acc.at[cidx.at[b]], add=True)
        return carry

    lax.fori_loop(0, NB, step, 0)
    plsc.subcore_barrier()

    @pl.when(s < WT)
    def _():
        pltpu.sync_copy(acc.at[pl.ds(s * 1000, 1000)],
                        out.at[c, pl.ds(s * 1000, 1000)])


@functools.partial(
    pl.kernel,
    out_type=jax.ShapeDtypeStruct((NC, N, 128), jnp.float32),
    mesh=_MESH,
    scratch_types=[
        pltpu.VMEM_SHARED((NACC, 128), jnp.float32),
        pltpu.VMEM((NB + 1, EB), jnp.int32),
        pltpu.VMEM((NB, EB), jnp.int32),
        pltpu.VMEM((EB, 128), jnp.float32),
    ],
)
def _agg_kernel(g2, rowp, colp, zer_hbm, out, acc, ridx, cidx, gbuf):
    _agg_body(g2, rowp, colp, zer_hbm, out, acc, ridx, cidx, gbuf)


_BT = 1000


def _prep_body(deg_ref, x_ref, g_ref):
    dinv = lax.rsqrt(deg_ref[0, :, 0:1] + deg_ref[1, :, 0:1] + 1.0)
    g_ref[...] = dinv * x_ref[...]


def _layer_body(relu, emit_g, deg_ref, s_ref, g_ref, res_ref, w_ref, *outs):
    dinv = lax.rsqrt(deg_ref[0, :, 0:1] + deg_ref[1, :, 0:1] + 1.0)
    sfull = jnp.concatenate([s_ref[0], s_ref[1]], axis=-1)
    agg = dinv * (sfull + g_ref[...])
    hv = lax.dot_general(agg, w_ref[...], (((1,), (1,)), ((), ())),
                         preferred_element_type=jnp.float32)
    if relu:
        hv = jnp.maximum(hv, 0.0)
    hv = hv + res_ref[...]
    outs[0][...] = hv
    if emit_g:
        outs[1][...] = dinv * hv


def _tc_prep(deg2, x):
    return pl.pallas_call(
        _prep_body,
        grid=(N // _BT,),
        in_specs=[
            pl.BlockSpec((NC, _BT, 128), lambda i: (0, i, 0)),
            pl.BlockSpec((_BT, D), lambda i: (i, 0)),
        ],
        out_specs=pl.BlockSpec((_BT, D), lambda i: (i, 0)),
        out_shape=jax.ShapeDtypeStruct((N, D), jnp.float32),
    )(deg2, x)


def _tc_layer(deg2, s2, g, res, w, relu, emit_g):
    n_out = 2 if emit_g else 1
    out_shape = [jax.ShapeDtypeStruct((N, D), jnp.float32)] * n_out
    out_specs = [pl.BlockSpec((_BT, D), lambda i: (i, 0))] * n_out
    res_ = pl.pallas_call(
        functools.partial(_layer_body, relu, emit_g),
        grid=(N // _BT,),
        in_specs=[
            pl.BlockSpec((NC, _BT, 128), lambda i: (0, i, 0)),
            pl.BlockSpec((NC, _BT, 128), lambda i: (0, i, 0)),
            pl.BlockSpec((_BT, D), lambda i: (i, 0)),
            pl.BlockSpec((_BT, D), lambda i: (i, 0)),
            pl.BlockSpec((D, D), lambda i: (0, 0)),
        ],
        out_specs=out_specs,
        out_shape=out_shape,
    )(deg2, s2, g, res, w)
    return res_ if emit_g else res_[0]


def kernel(x, edge_index, W0, W1):
    row = edge_index[0]
    col = edge_index[1]

    rowp = jnp.concatenate(
        [row, jnp.zeros((EPAD - E,), jnp.int32)]).reshape(NS, NB, EB)
    rowp = jnp.concatenate([rowp, jnp.zeros((NS, 1, EB), jnp.int32)], axis=1)
    colp = jnp.concatenate(
        [col, jnp.full((EPAD - E,), DUMMY, jnp.int32)]).reshape(NS, NB, EB)

    ones_hbm = jnp.ones((EB, 128), jnp.float32)
    zer128 = jnp.zeros((1000, 128), jnp.float32)

    deg2 = _deg_kernel(colp, ones_hbm, zer128)

    g0 = _tc_prep(deg2, x)
    s0 = _agg_kernel(g0.reshape(2 * N, 128), rowp, colp, zer128)
    h1, g1 = _tc_layer(deg2, s0, g0, x, W0, relu=True, emit_g=True)
    s1 = _agg_kernel(g1.reshape(2 * N, 128), rowp, colp, zer128)
    h2 = _tc_layer(deg2, s1, g1, h1, W1, relu=False, emit_g=False)
    return h2[None]
